# baseline (speedup 1.0000x reference)
import functools

import jax
import jax.numpy as jnp
from jax import lax
from jax.experimental import pallas as pl
from jax.experimental.pallas import tpu as pltpu
from jax.experimental.pallas import tpu_sc as plsc

B = 16384
F = 26
V = 100000
D = 16
DIN = 13
H = 64
DEEP = 64

TOT = B * F
NW = 32
CHUNK = 128
CPW = TOT // (NW * CHUNK)
STAGE = 26
NSTG = CPW // STAGE
SROWS = STAGE * CHUNK

CV = 1024
NCH = V // CV
TAIL = V - NCH * CV


def _tc_transpose(tab_t):
    RPF = V * D // 128

    def chunk(in_ref, out_ref, ff, v0, r0, n):
        x = in_ref[ff, :, pl.ds(v0, n)]
        y = jnp.transpose(x, (1, 0)).reshape(n // 8, 8, D)
        pieces = [y[:, j, :] for j in range(8)]
        out_ref[pl.ds(ff * RPF + r0, n * D // 128), :] = jnp.concatenate(
            pieces, axis=1)

    def body(in_ref, out_ref):
        for ff in range(2):
            def step(c, _):
                chunk(in_ref, out_ref, ff, c * CV, c * (CV * D // 128), CV)
                return 0
            lax.fori_loop(0, NCH, step, 0)
            chunk(in_ref, out_ref, ff, NCH * CV, NCH * (CV * D // 128), TAIL)

    return pl.pallas_call(
        body,
        grid=(F // 2,),
        in_specs=[pl.BlockSpec((2, D, V), lambda i: (i, 0, 0))],
        out_specs=pl.BlockSpec((2 * RPF, 128), lambda i: (i, 0)),
        out_shape=jax.ShapeDtypeStruct((F * RPF, 128), jnp.float32),
    )(tab_t)


def _sc_gather(cat_c, tab):
    mesh = plsc.VectorSubcoreMesh(core_axis_name="c", subcore_axis_name="s")

    @functools.partial(
        pl.kernel,
        mesh=mesh,
        compiler_params=pltpu.CompilerParams(use_tc_tiling_on_sc=False),
        out_type=jax.ShapeDtypeStruct((TOT, D), jnp.float32),
        scratch_types=[
            pltpu.VMEM((CPW, CHUNK), jnp.int32),
            pltpu.VMEM((SROWS, D), jnp.float32),
            pltpu.VMEM((SROWS, D), jnp.float32),
            pltpu.SemaphoreType.DMA,
            pltpu.SemaphoreType.DMA,
            pltpu.SemaphoreType.DMA,
            pltpu.SemaphoreType.DMA,
        ],
    )
    def k(cat_hbm, tab_hbm, out_hbm, idx_v, buf0, buf1, g0, g1, w0, w1):
        wid = lax.axis_index("s") * 2 + lax.axis_index("c")
        cbase = wid * CPW
        obase = wid * CPW * CHUNK

        pltpu.sync_copy(cat_hbm.at[pl.ds(cbase, CPW)], idx_v)

        bufs = (buf0, buf1)
        gsems = (g0, g1)
        wsems = (w0, w1)

        def add_offsets(r, _):
            for j in range(CHUNK // 16):
                gbase = (cbase + r) * CHUNK + j * 16
                f = lax.rem(gbase + lax.iota(jnp.int32, 16), F)
                sl = pl.ds(j * 16, 16)
                idx_v[r, sl] = idx_v[r, sl] + f * V
            return 0

        def stage_rows(s):
            return lax.fori_loop(s * STAGE, (s + 1) * STAGE, add_offsets, 0)

        def fire(s, g):
            return pltpu.make_async_copy(
                tab_hbm.at[idx_v.at[s * STAGE + g]],
                bufs[s % 2].at[pl.ds(g * CHUNK, CHUNK)],
                gsems[s % 2],
            )

        def wcopy(s):
            return pltpu.make_async_copy(
                bufs[s % 2],
                out_hbm.at[pl.ds(obase + s * SROWS, SROWS)],
                wsems[s % 2],
            )

        stage_rows(0)
        for s in range(NSTG):
            if s >= 2:
                wcopy(s - 2).wait()
            lax.fori_loop(0, STAGE, lambda g, _: (fire(s, g).start(), 0)[1], 0)
            if s + 1 < NSTG:
                stage_rows(s + 1)
            lax.fori_loop(0, STAGE, lambda g, _: (fire(s, g).wait(), 0)[1], 0)
            wcopy(s).start()
        wcopy(NSTG - 2).wait()
        wcopy(NSTG - 1).wait()

    return k(cat_c, tab)


def _tc_dense(cf, emb2, W1, b1, W2, b2, Wd_de, Wd_emb, bd, Wl_de, Wl_dp, wl_fm, bl):
    BLK = 2048
    grid = (B // BLK,)

    def body(cf_ref, emb_ref, w1_ref, b1_ref, w2_ref, b2_ref, wde_ref,
             wdem_ref, bd_ref, wl1_ref, wl2_ref, wlf_ref, bl_ref, out_ref):
        cf_blk = cf_ref[...]
        h = jnp.maximum(
            jnp.dot(cf_blk, w1_ref[...], preferred_element_type=jnp.float32)
            + b1_ref[...], 0.0)
        de = jnp.maximum(
            jnp.dot(h, w2_ref[...], preferred_element_type=jnp.float32)
            + b2_ref[...], 0.0)
        emb = emb_ref[...]
        deep = jnp.maximum(
            jnp.dot(de, wde_ref[...], preferred_element_type=jnp.float32)
            + jnp.dot(emb, wdem_ref[...], preferred_element_type=jnp.float32)
            + bd_ref[...], 0.0)
        s1 = (jnp.sum(de, axis=1, keepdims=True)
              + jnp.sum(emb, axis=1, keepdims=True))
        s2 = (jnp.sum(de * de, axis=1, keepdims=True)
              + jnp.sum(emb * emb, axis=1, keepdims=True))
        fm = 0.5 * (s1 * s1 - s2)
        out_ref[...] = (
            jnp.dot(de, wl1_ref[...], preferred_element_type=jnp.float32)
            + jnp.dot(deep, wl2_ref[...], preferred_element_type=jnp.float32)
            + fm * wlf_ref[...] + bl_ref[...])

    full = lambda shape: pl.BlockSpec(shape, lambda i: (0,) * len(shape))
    return pl.pallas_call(
        body,
        grid=grid,
        in_specs=[
            pl.BlockSpec((BLK, DIN), lambda i: (i, 0)),
            pl.BlockSpec((BLK, F * D), lambda i: (i, 0)),
            full((DIN, H)),
            full((1, H)),
            full((H, D)),
            full((1, D)),
            full((D, DEEP)),
            full((F * D, DEEP)),
            full((1, DEEP)),
            full((D, 1)),
            full((DEEP, 1)),
            full((1, 1)),
            full((1, 1)),
        ],
        out_specs=pl.BlockSpec((BLK, 1), lambda i: (i, 0)),
        out_shape=jax.ShapeDtypeStruct((B, 1), jnp.float32),
    )(cf, emb2, W1, b1, W2, b2, Wd_de, Wd_emb, bd, Wl_de, Wl_dp, wl_fm, bl)


def kernel(count_features, category_features, tables, W1, b1, W2, b2, Wd, bd, Wl, bl):
    cat_c = category_features.astype(jnp.int32).reshape(TOT // CHUNK, CHUNK)
    tab = _tc_transpose(tables.transpose(0, 2, 1)).reshape(F * V, D)
    emb_flat = _sc_gather(cat_c, tab)
    emb2 = emb_flat.reshape(B, F * D)
    logits = _tc_dense(
        count_features, emb2, W1, b1.reshape(1, H), W2, b2.reshape(1, D),
        Wd[:D], Wd[D:], bd.reshape(1, DEEP),
        Wl[:D], Wl[D:D + DEEP], Wl[D + DEEP:].reshape(1, 1), bl.reshape(1, 1))
    return logits


# speedup vs baseline: 1.9358x; 1.9358x over previous
import functools

import jax
import jax.numpy as jnp
from jax import lax
from jax.experimental import pallas as pl
from jax.experimental.pallas import tpu as pltpu
from jax.experimental.pallas import tpu_sc as plsc

B = 16384
F = 26
V = 100000
D = 16
DIN = 13
H = 64
DEEP = 64

TOT = B * F
NW = 32
CHUNK = 256
CPW = TOT // (NW * CHUNK)


def _sc_gather(cat_c, tab_w):
    mesh = plsc.VectorSubcoreMesh(core_axis_name="c", subcore_axis_name="s")

    @functools.partial(
        pl.kernel,
        mesh=mesh,
        compiler_params=pltpu.CompilerParams(use_tc_tiling_on_sc=False),
        out_type=jax.ShapeDtypeStruct((TOT * D,), jnp.float32),
        scratch_types=[
            pltpu.VMEM((CPW, CHUNK), jnp.int32),
            pltpu.VMEM((CHUNK * D // 128, 128), jnp.int32),
            pltpu.VMEM((CHUNK * D // 128, 128), jnp.int32),
            pltpu.VMEM((CHUNK * D,), jnp.float32),
            pltpu.VMEM((CHUNK * D,), jnp.float32),
            pltpu.SemaphoreType.DMA,
            pltpu.SemaphoreType.DMA,
        ],
    )
    def k(cat_hbm, tab_hbm, out_hbm, idx_v, ga, gb, oa, ob, ma, mb):
        wid = lax.axis_index("s") * 2 + lax.axis_index("c")
        cbase = wid * CPW
        wbase = wid * CPW * CHUNK * D

        pltpu.sync_copy(cat_hbm.at[pl.ds(cbase, CPW)], idx_v)

        lane = lax.iota(jnp.int32, 16)
        dword = lane * V

        def build(r, gbuf):
            e0 = (cbase + r) * CHUNK

            def group(q, _):
                vv = idx_v[r, pl.ds(q * 16, 16)]
                fv = lax.rem(e0 + q * 16 + lane, F)
                base = fv * (D * V) + vv
                for il in range(16):
                    b = jnp.take(base, jnp.full((16,), il, jnp.int32))
                    gbuf[2 * q + il // 8, pl.ds((il % 8) * 16, 16)] = b + dword
                return 0

            lax.fori_loop(0, CHUNK // 16, group, 0)

        def fire(gbuf, obuf, sem):
            def go(j, _):
                pltpu.make_async_copy(
                    tab_hbm.at[gbuf.at[j]],
                    obuf.at[pl.ds(j * 128, 128)],
                    sem,
                ).start()
                return 0

            lax.fori_loop(0, CHUNK * D // 128, go, 0)

        def drain(gbuf, obuf, sem):
            def go(j, _):
                pltpu.make_async_copy(
                    tab_hbm.at[gbuf.at[j]],
                    obuf.at[pl.ds(j * 128, 128)],
                    sem,
                ).wait()
                return 0

            lax.fori_loop(0, CHUNK * D // 128, go, 0)

        def write(r, obuf):
            pltpu.sync_copy(
                obuf, out_hbm.at[pl.ds(wbase + r * CHUNK * D, CHUNK * D)])

        build(0, ga)
        fire(ga, oa, ma)
        NP = CPW // 2

        def pair(p, _):
            r0 = 2 * p

            build(r0 + 1, gb)
            fire(gb, ob, mb)
            drain(ga, oa, ma)
            write(r0, oa)

            @pl.when(p + 1 < NP)
            def _():
                build(r0 + 2, ga)
                fire(ga, oa, ma)

            drain(gb, ob, mb)
            write(r0 + 1, ob)
            return 0

        lax.fori_loop(0, NP, pair, 0)

    return k(cat_c, tab_w)


def _tc_dense(cf, emb2, W1, b1, W2, b2, Wd_de, Wd_emb, bd, Wl_de, Wl_dp, wl_fm, bl):
    BLK = 2048
    grid = (B // BLK,)

    def body(cf_ref, emb_ref, w1_ref, b1_ref, w2_ref, b2_ref, wde_ref,
             wdem_ref, bd_ref, wl1_ref, wl2_ref, wlf_ref, bl_ref, out_ref):
        cf_blk = cf_ref[...]
        h = jnp.maximum(
            jnp.dot(cf_blk, w1_ref[...], preferred_element_type=jnp.float32)
            + b1_ref[...], 0.0)
        de = jnp.maximum(
            jnp.dot(h, w2_ref[...], preferred_element_type=jnp.float32)
            + b2_ref[...], 0.0)
        emb = emb_ref[...]
        deep = jnp.maximum(
            jnp.dot(de, wde_ref[...], preferred_element_type=jnp.float32)
            + jnp.dot(emb, wdem_ref[...], preferred_element_type=jnp.float32)
            + bd_ref[...], 0.0)
        s1 = (jnp.sum(de, axis=1, keepdims=True)
              + jnp.sum(emb, axis=1, keepdims=True))
        s2 = (jnp.sum(de * de, axis=1, keepdims=True)
              + jnp.sum(emb * emb, axis=1, keepdims=True))
        fm = 0.5 * (s1 * s1 - s2)
        out_ref[...] = (
            jnp.dot(de, wl1_ref[...], preferred_element_type=jnp.float32)
            + jnp.dot(deep, wl2_ref[...], preferred_element_type=jnp.float32)
            + fm * wlf_ref[...] + bl_ref[...])

    full = lambda shape: pl.BlockSpec(shape, lambda i: (0,) * len(shape))
    return pl.pallas_call(
        body,
        grid=grid,
        in_specs=[
            pl.BlockSpec((BLK, DIN), lambda i: (i, 0)),
            pl.BlockSpec((BLK, F * D), lambda i: (i, 0)),
            full((DIN, H)),
            full((1, H)),
            full((H, D)),
            full((1, D)),
            full((D, DEEP)),
            full((F * D, DEEP)),
            full((1, DEEP)),
            full((D, 1)),
            full((DEEP, 1)),
            full((1, 1)),
            full((1, 1)),
        ],
        out_specs=pl.BlockSpec((BLK, 1), lambda i: (i, 0)),
        out_shape=jax.ShapeDtypeStruct((B, 1), jnp.float32),
    )(cf, emb2, W1, b1, W2, b2, Wd_de, Wd_emb, bd, Wl_de, Wl_dp, wl_fm, bl)


def kernel(count_features, category_features, tables, W1, b1, W2, b2, Wd, bd, Wl, bl):
    cat_c = category_features.astype(jnp.int32).reshape(TOT // CHUNK, CHUNK)
    tab_w = tables.transpose(0, 2, 1).reshape(F * D * V)
    emb_flat = _sc_gather(cat_c, tab_w)
    emb2 = emb_flat.reshape(B, F * D)
    logits = _tc_dense(
        count_features, emb2, W1, b1.reshape(1, H), W2, b2.reshape(1, D),
        Wd[:D], Wd[D:], bd.reshape(1, DEEP),
        Wl[:D], Wl[D:D + DEEP], Wl[D + DEEP:].reshape(1, 1), bl.reshape(1, 1))
    return logits
